# baseline (speedup 1.0000x reference)
import jax
import jax.numpy as jnp
from jax import lax
from jax.experimental import pallas as pl
from jax.experimental.pallas import tpu as pltpu

_N = 320000
_D = 128
_S = 512
_B = 1280


def _body(x_ref, ids_ref, w1_ref, b1_ref, w2_ref, b2_ref, out_ref,
          acc_ref, cnt_ref):
    i = pl.program_id(0)
    nb = pl.num_programs(0)

    @pl.when(i == 0)
    def _init():
        acc_ref[...] = jnp.zeros_like(acc_ref)
        cnt_ref[...] = jnp.zeros_like(cnt_ref)

    x = x_ref[...]
    y = jnp.maximum(
        jnp.dot(x, w1_ref[...], preferred_element_type=jnp.float32)
        + b1_ref[...], 0.0)

    ids = ids_ref[0, 0, :]
    oh = (ids[:, None] == lax.broadcasted_iota(jnp.int32, (_B, _S), 1)
          ).astype(jnp.bfloat16)

    acc_ref[...] += lax.dot_general(
        oh, y.astype(jnp.bfloat16), (((0,), (0,)), ((), ())),
        preferred_element_type=jnp.float32)
    cnt_ref[...] += lax.dot_general(
        oh, jnp.ones((_B, 1), jnp.bfloat16), (((0,), (0,)), ((), ())),
        preferred_element_type=jnp.float32)

    @pl.when(i == nb - 1)
    def _finish():
        pooled = acc_ref[...] / jnp.maximum(cnt_ref[...], 1.0)
        out_ref[...] = (
            jnp.dot(pooled, w2_ref[...], preferred_element_type=jnp.float32)
            + b2_ref[...])


def kernel(input, batch, emb_weight, emb_bias, mlp_weight, mlp_bias):
    nb = _N // _B
    ids3 = batch.astype(jnp.int32).reshape(nb, 1, _B)
    b1 = emb_bias.reshape(1, _D)
    b2 = mlp_bias.reshape(1, _D)
    return pl.pallas_call(
        _body,
        grid=(nb,),
        in_specs=[
            pl.BlockSpec((_B, _D), lambda i: (i, 0)),
            pl.BlockSpec((1, 1, _B), lambda i: (i, 0, 0)),
            pl.BlockSpec((_D, _D), lambda i: (0, 0)),
            pl.BlockSpec((1, _D), lambda i: (0, 0)),
            pl.BlockSpec((_D, _D), lambda i: (0, 0)),
            pl.BlockSpec((1, _D), lambda i: (0, 0)),
        ],
        out_specs=pl.BlockSpec((_S, _D), lambda i: (0, 0)),
        out_shape=jax.ShapeDtypeStruct((_S, _D), jnp.float32),
        scratch_shapes=[
            pltpu.VMEM((_S, _D), jnp.float32),
            pltpu.VMEM((_S, 1), jnp.float32),
        ],
    )(input, ids3, emb_weight, b1, mlp_weight, b2)


# speedup vs baseline: 1.3452x; 1.3452x over previous
import jax
import jax.numpy as jnp
from jax import lax
from jax.experimental import pallas as pl
from jax.experimental.pallas import tpu as pltpu

_N = 320000
_D = 128
_S = 512
_B = 1280


def _body(x_ref, ids_ref, w1_ref, b1_ref, w2_ref, b2_ref, out_ref,
          acc_ref, cnt_ref):
    i = pl.program_id(0)
    nb = pl.num_programs(0)

    @pl.when(i == 0)
    def _init():
        acc_ref[...] = jnp.zeros_like(acc_ref)
        cnt_ref[...] = jnp.zeros_like(cnt_ref)

    x = x_ref[...]
    y = jnp.maximum(
        jnp.dot(x, w1_ref[...], preferred_element_type=jnp.float32)
        + b1_ref[...], 0.0)

    ids = ids_ref[0, 0, :].reshape(1, _B)
    oht = (ids == lax.broadcasted_iota(jnp.int32, (_S, _B), 0)
           ).astype(jnp.float32)

    acc_ref[...] += jnp.dot(oht, y, preferred_element_type=jnp.float32)
    cnt_ref[...] += jnp.sum(oht, axis=1, keepdims=True)

    @pl.when(i == nb - 1)
    def _finish():
        pooled = acc_ref[...] / jnp.maximum(cnt_ref[...], 1.0)
        out_ref[...] = (
            jnp.dot(pooled, w2_ref[...], preferred_element_type=jnp.float32)
            + b2_ref[...])


def kernel(input, batch, emb_weight, emb_bias, mlp_weight, mlp_bias):
    nb = _N // _B
    ids3 = batch.astype(jnp.int32).reshape(nb, 1, _B)
    b1 = emb_bias.reshape(1, _D)
    b2 = mlp_bias.reshape(1, _D)
    return pl.pallas_call(
        _body,
        grid=(nb,),
        in_specs=[
            pl.BlockSpec((_B, _D), lambda i: (i, 0)),
            pl.BlockSpec((1, 1, _B), lambda i: (i, 0, 0)),
            pl.BlockSpec((_D, _D), lambda i: (0, 0)),
            pl.BlockSpec((1, _D), lambda i: (0, 0)),
            pl.BlockSpec((_D, _D), lambda i: (0, 0)),
            pl.BlockSpec((1, _D), lambda i: (0, 0)),
        ],
        out_specs=pl.BlockSpec((_S, _D), lambda i: (0, 0)),
        out_shape=jax.ShapeDtypeStruct((_S, _D), jnp.float32),
        scratch_shapes=[
            pltpu.VMEM((_S, _D), jnp.float32),
            pltpu.VMEM((_S, 1), jnp.float32),
        ],
    )(input, ids3, emb_weight, b1, mlp_weight, b2)


# speedup vs baseline: 1.3473x; 1.0015x over previous
import jax
import jax.numpy as jnp
from jax import lax
from jax.experimental import pallas as pl
from jax.experimental.pallas import tpu as pltpu

_N = 320000
_D = 128
_S = 512
_B = 1280


def _body(x_ref, ids_ref, w1_ref, b1_ref, w2_ref, b2_ref, out_ref,
          acc_ref, cnt_ref):
    i = pl.program_id(0)
    nb = pl.num_programs(0)

    @pl.when(i == 0)
    def _init():
        acc_ref[...] = jnp.zeros_like(acc_ref)
        cnt_ref[...] = jnp.zeros_like(cnt_ref)

    x = x_ref[...]
    y = jnp.maximum(
        jnp.dot(x, w1_ref[...], preferred_element_type=jnp.float32)
        + b1_ref[...], 0.0)

    ids = ids_ref[0, 0, :].reshape(1, _B)
    eq = ids == lax.broadcasted_iota(jnp.int32, (_S, _B), 0)
    oht = eq.astype(jnp.bfloat16)

    acc_ref[...] += jnp.dot(oht, y.astype(jnp.bfloat16),
                            preferred_element_type=jnp.float32)
    cnt_ref[...] += jnp.sum(eq.astype(jnp.float32), axis=1, keepdims=True)

    @pl.when(i == nb - 1)
    def _finish():
        pooled = acc_ref[...] / jnp.maximum(cnt_ref[...], 1.0)
        out_ref[...] = (
            jnp.dot(pooled, w2_ref[...], preferred_element_type=jnp.float32)
            + b2_ref[...])


def kernel(input, batch, emb_weight, emb_bias, mlp_weight, mlp_bias):
    nb = _N // _B
    ids3 = batch.astype(jnp.int32).reshape(nb, 1, _B)
    b1 = emb_bias.reshape(1, _D)
    b2 = mlp_bias.reshape(1, _D)
    return pl.pallas_call(
        _body,
        grid=(nb,),
        in_specs=[
            pl.BlockSpec((_B, _D), lambda i: (i, 0)),
            pl.BlockSpec((1, 1, _B), lambda i: (i, 0, 0)),
            pl.BlockSpec((_D, _D), lambda i: (0, 0)),
            pl.BlockSpec((1, _D), lambda i: (0, 0)),
            pl.BlockSpec((_D, _D), lambda i: (0, 0)),
            pl.BlockSpec((1, _D), lambda i: (0, 0)),
        ],
        out_specs=pl.BlockSpec((_S, _D), lambda i: (0, 0)),
        out_shape=jax.ShapeDtypeStruct((_S, _D), jnp.float32),
        scratch_shapes=[
            pltpu.VMEM((_S, _D), jnp.float32),
            pltpu.VMEM((_S, 1), jnp.float32),
        ],
    )(input, ids3, emb_weight, b1, mlp_weight, b2)
